# baseline (speedup 1.0000x reference)
import functools

import jax
import jax.numpy as jnp
from jax import lax
from jax.experimental import pallas as pl
from jax.experimental.pallas import tpu as pltpu
from jax.experimental.pallas import tpu_sc as plsc

B = 16384
L = 20
E = 64
NC = 2
NS = 16
NW = NC * NS
BPW = B // NW
C = 16
STEPS = BPW // C


def _body(tid_hbm, cid_hbm, temb_hbm, cemb_hbm, out_hbm,
          tidx_v, cidx_v, h_v, u_v, ob_v, sem_h, sem_u):
    wid = lax.axis_index("s") * NC + lax.axis_index("c")
    liota = lax.iota(jnp.int32, 16)
    rowbase = [liota * L + l for l in range(L)]

    def step_fn(step, _):
        b0 = wid * BPW + step * C
        pltpu.sync_copy(tid_hbm.at[pl.ds(b0, C)], tidx_v)
        pltpu.sync_copy(cid_hbm.at[pl.ds(b0 * L, C * L)], cidx_v)
        cp_h = pltpu.async_copy(temb_hbm.at[tidx_v], h_v, sem_h)
        cp_u = pltpu.async_copy(cemb_hbm.at[cidx_v], u_v, sem_u)
        cp_h.wait()
        cp_u.wait()

        def estep(e, accs):
            ecol = jnp.full((16,), e, jnp.int32)
            he = plsc.load_gather(h_v, [liota, ecol])
            return tuple(
                acc + he * plsc.load_gather(u_v, [rowbase[l], ecol])
                for l, acc in enumerate(accs)
            )

        accs = lax.fori_loop(
            0, E, estep,
            tuple(jnp.zeros((16,), jnp.float32) for _ in range(L)))

        for l in range(L):
            sig = 1.0 / (1.0 + jnp.exp(-accs[l]))
            plsc.store_scatter(ob_v, [rowbase[l]], sig)

        pltpu.sync_copy(ob_v, out_hbm.at[pl.ds(b0 * L, C * L)])
        return ()

    lax.fori_loop(0, STEPS, step_fn, ())


@jax.jit
def _run(tid, cid, temb, cemb):
    mesh = plsc.VectorSubcoreMesh(
        core_axis_name="c", subcore_axis_name="s",
        num_cores=NC, num_subcores=NS)
    f = pl.kernel(
        _body,
        out_type=jax.ShapeDtypeStruct((B * L,), jnp.float32),
        mesh=mesh,
        scratch_types=[
            pltpu.VMEM((C,), jnp.int32),
            pltpu.VMEM((C * L,), jnp.int32),
            pltpu.VMEM((C, E), jnp.float32),
            pltpu.VMEM((C * L, E), jnp.float32),
            pltpu.VMEM((C * L,), jnp.float32),
            pltpu.SemaphoreType.DMA,
            pltpu.SemaphoreType.DMA,
        ],
        compiler_params=pltpu.CompilerParams(
            needs_layout_passes=False, use_tc_tiling_on_sc=False),
    )
    return f(tid, cid, temb, cemb)


def kernel(target_word_id, context_word_ids, target_embeddings,
           context_embeddings):
    tid = target_word_id.reshape(-1).astype(jnp.int32)
    cid = context_word_ids.reshape(-1).astype(jnp.int32)
    out = _run(tid, cid, target_embeddings, context_embeddings)
    return out.reshape(B, L)
